# baseline (speedup 1.0000x reference)
import functools

import jax
import jax.numpy as jnp
from jax import lax
from jax.experimental import pallas as pl
from jax.experimental.pallas import tpu as pltpu
from jax.experimental.pallas import tpu_sc as plsc

_info = plsc.get_sparse_core_info()
_NS = _info.num_subcores


@functools.lru_cache(maxsize=None)
def _make_gather(batch: int, nchunk: int):
    assert batch % _NS == 0
    b_per_w = batch // _NS
    assert b_per_w % nchunk == 0
    c = b_per_w // nchunk
    assert c % 8 == 0
    mesh = plsc.VectorSubcoreMesh(core_axis_name="c", subcore_axis_name="s",
                                  num_cores=1)

    @functools.partial(
        pl.kernel,
        mesh=mesh,
        out_type=jax.ShapeDtypeStruct((batch,), jnp.float32),
        scratch_types=[
            pltpu.VMEM((b_per_w,), jnp.int32),
            pltpu.VMEM((b_per_w,), jnp.float32),
            pltpu.SemaphoreType.DMA,
            pltpu.SemaphoreType.DMA,
            pltpu.SemaphoreType.DMA,
        ],
    )
    def gather_kernel(idx_hbm, table_hbm, out_hbm, idx_v, vals_v,
                      sem_i, sem_g, sem_o):
        sid = lax.axis_index("s")
        base = sid * b_per_w
        idx_cp = [
            pltpu.async_copy(idx_hbm.at[pl.ds(base + j * c, c)],
                             idx_v.at[pl.ds(j * c, c)], sem_i)
            for j in range(nchunk)
        ]
        g_cp = []
        for j in range(nchunk):
            idx_cp[j].wait()
            g_cp.append(
                pltpu.async_copy(table_hbm.at[idx_v.at[pl.ds(j * c, c)]],
                                 vals_v.at[pl.ds(j * c, c)], sem_g))
        o_cp = []
        for j in range(nchunk):
            g_cp[j].wait()
            o_cp.append(
                pltpu.async_copy(vals_v.at[pl.ds(j * c, c)],
                                 out_hbm.at[pl.ds(base + j * c, c)], sem_o))
        for j in range(nchunk):
            o_cp[j].wait()

    return gather_kernel


def kernel(id, map_param):
    idx = id.astype(jnp.int32)
    return _make_gather(idx.shape[0], 4)(idx, map_param)


# speedup vs baseline: 1.0103x; 1.0103x over previous
import functools

import jax
import jax.numpy as jnp
from jax import lax
from jax.experimental import pallas as pl
from jax.experimental.pallas import tpu as pltpu
from jax.experimental.pallas import tpu_sc as plsc

_info = plsc.get_sparse_core_info()
_NS = _info.num_subcores


@functools.lru_cache(maxsize=None)
def _make_gather(batch: int, first_chunk: int):
    assert batch % _NS == 0
    b_per_w = batch // _NS
    sizes = [first_chunk, b_per_w - first_chunk]
    offs = [0, first_chunk]
    assert all(s > 0 and s % 8 == 0 for s in sizes)
    mesh = plsc.VectorSubcoreMesh(core_axis_name="c", subcore_axis_name="s",
                                  num_cores=1)

    @functools.partial(
        pl.kernel,
        mesh=mesh,
        out_type=jax.ShapeDtypeStruct((batch,), jnp.float32),
        scratch_types=[
            pltpu.VMEM((b_per_w,), jnp.int32),
            pltpu.VMEM((b_per_w,), jnp.float32),
            pltpu.SemaphoreType.DMA,
            pltpu.SemaphoreType.DMA,
            pltpu.SemaphoreType.DMA,
        ],
    )
    def gather_kernel(idx_hbm, table_hbm, out_hbm, idx_v, vals_v,
                      sem_i, sem_g, sem_o):
        sid = lax.axis_index("s")
        base = sid * b_per_w
        idx_cp = [
            pltpu.async_copy(idx_hbm.at[pl.ds(base + o, s)],
                             idx_v.at[pl.ds(o, s)], sem_i)
            for o, s in zip(offs, sizes)
        ]
        g_cp = []
        for j, (o, s) in enumerate(zip(offs, sizes)):
            idx_cp[j].wait()
            g_cp.append(
                pltpu.async_copy(table_hbm.at[idx_v.at[pl.ds(o, s)]],
                                 vals_v.at[pl.ds(o, s)], sem_g))
        o_cp = []
        for j, (o, s) in enumerate(zip(offs, sizes)):
            g_cp[j].wait()
            o_cp.append(
                pltpu.async_copy(vals_v.at[pl.ds(o, s)],
                                 out_hbm.at[pl.ds(base + o, s)], sem_o))
        for cp in o_cp:
            cp.wait()

    return gather_kernel


def kernel(id, map_param):
    idx = id.astype(jnp.int32)
    return _make_gather(idx.shape[0], 128)(idx, map_param)
